# baseline (speedup 1.0000x reference)
import functools

import jax
import jax.numpy as jnp
from jax import lax
from jax.experimental import pallas as pl
from jax.experimental.pallas import tpu as pltpu
from jax.experimental.pallas import tpu_sc as plsc

_NUM_CORES = 2
_NUM_SUBCORES = 16
_NUM_WORKERS = _NUM_CORES * _NUM_SUBCORES
_LANES = 16


def _proj_body(e_ref, w_ref, o_ref):
    e = e_ref[...]
    w = w_ref[...]
    e_hi = e.astype(jnp.bfloat16)
    e_lo = (e - e_hi.astype(jnp.float32)).astype(jnp.bfloat16)
    w_hi = w.astype(jnp.bfloat16)
    w_lo = (w - w_hi.astype(jnp.float32)).astype(jnp.bfloat16)

    def mm(a, b):
        return jnp.dot(a, b, preferred_element_type=jnp.float32)

    o_ref[...] = mm(e_hi, w_hi) + mm(e_hi, w_lo) + mm(e_lo, w_hi)


def _project(vocab, dim, pdim):
    bm = 8000
    assert vocab % bm == 0
    return pl.pallas_call(
        _proj_body,
        grid=(vocab // bm,),
        in_specs=[
            pl.BlockSpec((bm, dim), lambda i: (i, 0)),
            pl.BlockSpec((dim, pdim), lambda i: (0, 0)),
        ],
        out_specs=pl.BlockSpec((bm, pdim), lambda i: (i, 0)),
        out_shape=jax.ShapeDtypeStruct((vocab, pdim), jnp.float32),
    )


def _make_pooled(seq, batch, pdim, half):
    bpw = (batch // 2) // _NUM_SUBCORES
    mesh = plsc.VectorSubcoreMesh(core_axis_name="c", subcore_axis_name="s",
                                  num_cores=1, num_subcores=_NUM_SUBCORES)

    @functools.partial(
        pl.kernel,
        mesh=mesh,
        out_type=jax.ShapeDtypeStruct((batch // 2, pdim), jnp.float32),
        compiler_params=pltpu.CompilerParams(use_tc_tiling_on_sc=False),
        scratch_types=[
            pltpu.VMEM((seq, bpw), jnp.int32),
            pltpu.VMEM((bpw, pdim), jnp.float32),
            pltpu.VMEM((bpw, pdim), jnp.float32),
            pltpu.VMEM((bpw, pdim), jnp.float32),
            pltpu.VMEM((_LANES,), jnp.float32),
            pltpu.SemaphoreType.DMA,
            pltpu.SemaphoreType.DMA,
        ],
    )
    def pooled(text_hbm, ew_hbm, bias_hbm, out_hbm, idx_v, rows0, rows1,
               acc_v, bias_v, sem0, sem1):
        w = lax.axis_index("s")
        b0 = half * (batch // 2) + w * bpw

        pltpu.sync_copy(text_hbm.at[:, pl.ds(b0, bpw)], idx_v)
        pltpu.sync_copy(bias_hbm, bias_v)

        def gather(s, buf, sem):
            return pltpu.make_async_copy(ew_hbm.at[idx_v.at[s]], buf, sem)

        def accumulate(buf):
            @pl.loop(0, bpw, step=4)
            def _(i):
                for d in range(4):
                    acc_v[i + d, :] = acc_v[i + d, :] + buf[i + d, :]

        @pl.loop(0, bpw, step=4)
        def _(i):
            for d in range(4):
                acc_v[i + d, :] = jnp.zeros((_LANES,), jnp.float32)

        gather(0, rows0, sem0).start()

        @pl.loop(0, seq, step=2)
        def _(s):
            gather(s, rows0, sem0).wait()
            gather(s + 1, rows1, sem1).start()
            accumulate(rows0)
            gather(s + 1, rows1, sem1).wait()

            @pl.when(s + 2 < seq)
            def _():
                gather(s + 2, rows0, sem0).start()

            accumulate(rows1)

        @pl.loop(0, bpw, step=4)
        def _(i):
            for d in range(4):
                acc_v[i + d, :] = acc_v[i + d, :] + bias_v[:]

        pltpu.sync_copy(acc_v, out_hbm.at[pl.ds(w * bpw, bpw)])

    return pooled


def kernel(text, embedding_table, fc_w, fc_b):
    seq, batch = text.shape
    vocab, dim = embedding_table.shape
    out_dim = fc_w.shape[0]
    pdim = _LANES

    wp = jnp.zeros((dim, pdim), jnp.float32).at[:, :out_dim].set(fc_w.T / seq)
    bias16 = jnp.zeros((pdim,), jnp.float32).at[:out_dim].set(fc_b)

    ew = _project(vocab, dim, pdim)(embedding_table, wp)
    lo = _make_pooled(seq, batch, pdim, 0)(text, ew, bias16)
    hi = _make_pooled(seq, batch, pdim, 1)(text, ew, bias16)
    pooled = jnp.concatenate([lo, hi], axis=0)
    return pooled[:, :out_dim]


# speedup vs baseline: 2.8731x; 2.8731x over previous
import functools

import jax
import jax.numpy as jnp
from jax import lax
from jax.experimental import pallas as pl
from jax.experimental.pallas import tpu as pltpu
from jax.experimental.pallas import tpu_sc as plsc

_NUM_CORES = 2
_NUM_SUBCORES = 16
_NUM_WORKERS = _NUM_CORES * _NUM_SUBCORES
_LANES = 16
_PACK = 8
_CHUNK_BITS = 17
_CHUNK = 1 << _CHUNK_BITS


def _proj_body(et_refs, w_ref, o_ref):
    w = w_ref[...]
    w_hi = w.astype(jnp.bfloat16)
    w_lo = (w - w_hi.astype(jnp.float32)).astype(jnp.bfloat16)
    for j, et_ref in enumerate(et_refs):
        et = et_ref[...]
        e_hi = et.astype(jnp.bfloat16)
        e_lo = (et - e_hi.astype(jnp.float32)).astype(jnp.bfloat16)

        def mm(a, b):
            return jax.lax.dot_general(
                a, b, (((0,), (0,)), ((), ())),
                preferred_element_type=jnp.float32)

        z = mm(e_hi, w_hi) + mm(e_hi, w_lo) + mm(e_lo, w_hi)
        o_ref[:, j * _LANES:(j + 1) * _LANES] = z


def _project(vocab, dim, pdim):
    bn = 4096
    nblk = _CHUNK // bn
    last_blk = (vocab - 1) // bn

    def body(*refs):
        _proj_body(refs[:_PACK], refs[_PACK], refs[_PACK + 1])

    def make_imap(j):
        def imap(i):
            return (0, jnp.minimum(j * nblk + i, last_blk))
        return imap

    return pl.pallas_call(
        body,
        grid=(nblk,),
        in_specs=[
            pl.BlockSpec((dim, bn), make_imap(j)) for j in range(_PACK)
        ] + [pl.BlockSpec((dim, pdim), lambda i: (0, 0))],
        out_specs=pl.BlockSpec((bn, _PACK * pdim), lambda i: (i, 0)),
        out_shape=jax.ShapeDtypeStruct((_CHUNK, _PACK * pdim), jnp.float32),
    )


def _make_pooled(seq, batch, pdim):
    bpw = batch // _NUM_WORKERS
    mesh = plsc.VectorSubcoreMesh(core_axis_name="c", subcore_axis_name="s")

    @functools.partial(
        pl.kernel,
        mesh=mesh,
        out_type=jax.ShapeDtypeStruct((batch, pdim), jnp.float32),
        compiler_params=pltpu.CompilerParams(use_tc_tiling_on_sc=False),
        scratch_types=[
            pltpu.VMEM((seq, bpw), jnp.int32),
            pltpu.VMEM((bpw, pdim), jnp.float32),
            pltpu.VMEM((bpw, pdim), jnp.float32),
            pltpu.VMEM((bpw, pdim), jnp.float32),
            pltpu.VMEM((_LANES,), jnp.float32),
            pltpu.SemaphoreType.DMA,
            pltpu.SemaphoreType.DMA,
        ],
    )
    def pooled(text_hbm, ew_hbm, bias_hbm, out_hbm, idx_v, rows0, rows1,
               acc_v, bias_v, sem0, sem1):
        w = lax.axis_index("s") * _NUM_CORES + lax.axis_index("c")
        b0 = w * bpw

        pltpu.sync_copy(text_hbm.at[:, pl.ds(b0, bpw)], idx_v)
        pltpu.sync_copy(bias_hbm, bias_v)

        @pl.loop(0, seq)
        def _(s):
            for c in range(bpw // _LANES):
                sl = pl.ds(c * _LANES, _LANES)
                i = idx_v[s, sl]
                q = jax.lax.shift_right_logical(i, _CHUNK_BITS)
                r = jax.lax.bitwise_and(i, _CHUNK - 1)
                idx_v[s, sl] = jax.lax.shift_left(r, 3) + q

        def gather(s, buf, sem):
            return pltpu.make_async_copy(ew_hbm.at[idx_v.at[s]], buf, sem)

        def accumulate(buf):
            @pl.loop(0, bpw, step=4)
            def _(i):
                for d in range(4):
                    acc_v[i + d, :] = acc_v[i + d, :] + buf[i + d, :]

        @pl.loop(0, bpw, step=4)
        def _(i):
            for d in range(4):
                acc_v[i + d, :] = jnp.zeros((_LANES,), jnp.float32)

        gather(0, rows0, sem0).start()

        @pl.loop(0, seq, step=2)
        def _(s):
            gather(s, rows0, sem0).wait()
            gather(s + 1, rows1, sem1).start()
            accumulate(rows0)
            gather(s + 1, rows1, sem1).wait()

            @pl.when(s + 2 < seq)
            def _():
                gather(s + 2, rows0, sem0).start()

            accumulate(rows1)

        @pl.loop(0, bpw, step=4)
        def _(i):
            for d in range(4):
                acc_v[i + d, :] = acc_v[i + d, :] + bias_v[:]

        pltpu.sync_copy(acc_v, out_hbm.at[pl.ds(b0, bpw)])

    return pooled


def kernel(text, embedding_table, fc_w, fc_b):
    seq, batch = text.shape
    vocab, dim = embedding_table.shape
    out_dim = fc_w.shape[0]
    pdim = _LANES

    wp = jnp.zeros((dim, pdim), jnp.float32).at[:, :out_dim].set(fc_w.T / seq)
    bias16 = jnp.zeros((pdim,), jnp.float32).at[:out_dim].set(fc_b)

    et = embedding_table.T
    ew128 = _project(vocab, dim, pdim)(*([et] * _PACK), wp)
    ew = ew128.reshape(_PACK * _CHUNK, pdim)
    pooled = _make_pooled(seq, batch, pdim)(text, ew, bias16)
    return pooled[:, :out_dim]


# speedup vs baseline: 3.7307x; 1.2985x over previous
import functools

import jax
import jax.numpy as jnp
from jax import lax
from jax.experimental import pallas as pl
from jax.experimental.pallas import tpu as pltpu
from jax.experimental.pallas import tpu_sc as plsc

_NUM_CORES = 2
_NUM_SUBCORES = 16
_NUM_WORKERS = _NUM_CORES * _NUM_SUBCORES
_LANES = 16
_PACK = 8
_CHUNK_BITS = 17
_CHUNK = 1 << _CHUNK_BITS


def _proj_body(et_refs, w_ref, o_ref):
    w = w_ref[...]
    w_hi = w.astype(jnp.bfloat16)
    w_lo = (w - w_hi.astype(jnp.float32)).astype(jnp.bfloat16)
    for j, et_ref in enumerate(et_refs):
        et = et_ref[...]
        e_hi = et.astype(jnp.bfloat16)
        e_lo = (et - e_hi.astype(jnp.float32)).astype(jnp.bfloat16)

        def mm(a, b):
            return jax.lax.dot_general(
                a, b, (((0,), (0,)), ((), ())),
                preferred_element_type=jnp.float32)

        z = mm(e_hi, w_hi) + mm(e_hi, w_lo) + mm(e_lo, w_hi)
        o_ref[:, j * _LANES:(j + 1) * _LANES] = z


def _project(vocab, dim, pdim):
    bn = 4096
    nblk = _CHUNK // bn
    last_blk = (vocab - 1) // bn

    def body(*refs):
        _proj_body(refs[:_PACK], refs[_PACK], refs[_PACK + 1])

    def make_imap(j):
        def imap(i):
            return (0, jnp.minimum(j * nblk + i, last_blk))
        return imap

    return pl.pallas_call(
        body,
        grid=(nblk,),
        in_specs=[
            pl.BlockSpec((dim, bn), make_imap(j)) for j in range(_PACK)
        ] + [pl.BlockSpec((dim, pdim), lambda i: (0, 0))],
        out_specs=pl.BlockSpec((bn, _PACK * pdim), lambda i: (i, 0)),
        out_shape=jax.ShapeDtypeStruct((_CHUNK, _PACK * pdim), jnp.float32),
    )


def _make_pooled(seq, batch, pdim):
    bpw = batch // _NUM_WORKERS
    mesh = plsc.VectorSubcoreMesh(core_axis_name="c", subcore_axis_name="s")

    @functools.partial(
        pl.kernel,
        mesh=mesh,
        out_type=jax.ShapeDtypeStruct((batch, pdim), jnp.float32),
        compiler_params=pltpu.CompilerParams(use_tc_tiling_on_sc=False),
        scratch_types=[
            pltpu.VMEM((seq, bpw), jnp.int32),
        ] + [pltpu.VMEM((bpw, pdim), jnp.float32) for _ in range(9)] + [
            pltpu.VMEM((_LANES,), jnp.float32),
            pltpu.SemaphoreType.DMA,
            pltpu.SemaphoreType.DMA,
        ],
    )
    def pooled(text_hbm, ew_hbm, bias_hbm, out_hbm, idx_v,
               a0, a1, a2, a3, c0, c1, c2, c3, acc_v, bias_v, sem0, sem1):
        grp_a = (a0, a1, a2, a3)
        grp_b = (c0, c1, c2, c3)
        w = lax.axis_index("s") * _NUM_CORES + lax.axis_index("c")
        b0 = w * bpw

        pltpu.sync_copy(text_hbm.at[:, pl.ds(b0, bpw)], idx_v)
        pltpu.sync_copy(bias_hbm, bias_v)

        @pl.loop(0, seq)
        def _(s):
            for c in range(bpw // _LANES):
                sl = pl.ds(c * _LANES, _LANES)
                i = idx_v[s, sl]
                q = jax.lax.shift_right_logical(i, _CHUNK_BITS)
                r = jax.lax.bitwise_and(i, _CHUNK - 1)
                idx_v[s, sl] = jax.lax.shift_left(r, 3) + q

        def gather(s, buf, sem):
            return pltpu.make_async_copy(ew_hbm.at[idx_v.at[s]], buf, sem)

        def fire(s, bufs, sem):
            for b in range(4):
                gather(s + b, bufs[b], sem).start()

        def drain_acc(s, bufs, sem):
            for b in range(4):
                gather(s + b, bufs[b], sem).wait()

            @pl.loop(0, bpw, step=2)
            def _(i):
                for d in range(2):
                    acc_v[i + d, :] = acc_v[i + d, :] + (
                        (bufs[0][i + d, :] + bufs[1][i + d, :])
                        + (bufs[2][i + d, :] + bufs[3][i + d, :]))

        @pl.loop(0, bpw, step=4)
        def _(i):
            for d in range(4):
                acc_v[i + d, :] = jnp.zeros((_LANES,), jnp.float32)

        fire(0, grp_a, sem0)
        fire(4, grp_b, sem1)

        @pl.loop(0, seq, step=8)
        def _(s):
            drain_acc(s, grp_a, sem0)

            @pl.when(s + 8 < seq)
            def _():
                fire(s + 8, grp_a, sem0)

            drain_acc(s + 4, grp_b, sem1)

            @pl.when(s + 12 < seq)
            def _():
                fire(s + 12, grp_b, sem1)

        @pl.loop(0, bpw, step=4)
        def _(i):
            for d in range(4):
                acc_v[i + d, :] = acc_v[i + d, :] + bias_v[:]

        pltpu.sync_copy(acc_v, out_hbm.at[pl.ds(b0, bpw)])

    return pooled


def kernel(text, embedding_table, fc_w, fc_b):
    seq, batch = text.shape
    vocab, dim = embedding_table.shape
    out_dim = fc_w.shape[0]
    pdim = _LANES

    wp = jnp.zeros((dim, pdim), jnp.float32).at[:, :out_dim].set(fc_w.T / seq)
    bias16 = jnp.zeros((pdim,), jnp.float32).at[:out_dim].set(fc_b)

    et = embedding_table.T
    ew128 = _project(vocab, dim, pdim)(*([et] * _PACK), wp)
    ew = ew128.reshape(_PACK * _CHUNK, pdim)
    pooled = _make_pooled(seq, batch, pdim)(text, ew, bias16)
    return pooled[:, :out_dim]


# speedup vs baseline: 5.7877x; 1.5514x over previous
import functools

import jax
import jax.numpy as jnp
from jax import lax
from jax.experimental import pallas as pl
from jax.experimental.pallas import tpu as pltpu
from jax.experimental.pallas import tpu_sc as plsc

_NUM_CORES = 2
_NUM_SUBCORES = 16
_NUM_WORKERS = _NUM_CORES * _NUM_SUBCORES
_LANES = 16
_PACK = 8
_CHUNK_BITS = 17
_CHUNK = 1 << _CHUNK_BITS


def _proj_stage1(vocab, dim, pdim):
    bn = 16384
    nblk = -(-vocab // bn)

    def body(w_ref, et_ref, o_ref):
        o_ref[...] = jax.lax.dot_general(
            w_ref[...], et_ref[...], (((0,), (0,)), ((), ())),
            preferred_element_type=jnp.float32)

    return pl.pallas_call(
        body,
        grid=(nblk,),
        in_specs=[
            pl.BlockSpec((dim, pdim), lambda i: (0, 0)),
            pl.BlockSpec((dim, bn), lambda i: (0, i)),
        ],
        out_specs=pl.BlockSpec((pdim, bn), lambda i: (0, i)),
        out_shape=jax.ShapeDtypeStruct((pdim, vocab), jnp.float32),
    )


def _proj_stage2(vocab, pdim):
    br = 4096
    nblk = _CHUNK // br
    last_blk = (vocab - 1) // br

    def body(*refs):
        o_ref = refs[_PACK]
        zc = jnp.concatenate([refs[j][...] for j in range(_PACK)], axis=0)
        o_ref[...] = zc.T

    def make_imap(j):
        def imap(i):
            return (0, jnp.minimum(j * nblk + i, last_blk))
        return imap

    return pl.pallas_call(
        body,
        grid=(nblk,),
        in_specs=[
            pl.BlockSpec((pdim, br), make_imap(j)) for j in range(_PACK)
        ],
        out_specs=pl.BlockSpec((br, _PACK * _LANES), lambda i: (i, 0)),
        out_shape=jax.ShapeDtypeStruct((_CHUNK, _PACK * _LANES), jnp.float32),
    )


def _make_pooled(seq, batch, pdim):
    bpw = batch // _NUM_WORKERS
    mesh = plsc.VectorSubcoreMesh(core_axis_name="c", subcore_axis_name="s")

    @functools.partial(
        pl.kernel,
        mesh=mesh,
        out_type=jax.ShapeDtypeStruct((batch, pdim), jnp.float32),
        compiler_params=pltpu.CompilerParams(use_tc_tiling_on_sc=False),
        scratch_types=[
            pltpu.VMEM((seq, bpw), jnp.int32),
        ] + [pltpu.VMEM((bpw, pdim), jnp.float32) for _ in range(9)] + [
            pltpu.VMEM((_LANES,), jnp.float32),
            pltpu.SemaphoreType.DMA,
            pltpu.SemaphoreType.DMA,
        ],
    )
    def pooled(text_hbm, ew_hbm, bias_hbm, out_hbm, idx_v,
               a0, a1, a2, a3, c0, c1, c2, c3, acc_v, bias_v, sem0, sem1):
        grp_a = (a0, a1, a2, a3)
        grp_b = (c0, c1, c2, c3)
        w = lax.axis_index("s") * _NUM_CORES + lax.axis_index("c")
        b0 = w * bpw

        pltpu.sync_copy(text_hbm.at[:, pl.ds(b0, bpw)], idx_v)
        pltpu.sync_copy(bias_hbm, bias_v)

        @pl.loop(0, seq)
        def _(s):
            for c in range(bpw // _LANES):
                sl = pl.ds(c * _LANES, _LANES)
                i = idx_v[s, sl]
                q = jax.lax.shift_right_logical(i, _CHUNK_BITS)
                r = jax.lax.bitwise_and(i, _CHUNK - 1)
                idx_v[s, sl] = jax.lax.shift_left(r, 3) + q

        def gather(s, buf, sem):
            return pltpu.make_async_copy(ew_hbm.at[idx_v.at[s]], buf, sem)

        def fire(s, bufs, sem):
            for b in range(4):
                gather(s + b, bufs[b], sem).start()

        def drain_acc(s, bufs, sem):
            for b in range(4):
                gather(s + b, bufs[b], sem).wait()

            @pl.loop(0, bpw, step=2)
            def _(i):
                for d in range(2):
                    acc_v[i + d, :] = acc_v[i + d, :] + (
                        (bufs[0][i + d, :] + bufs[1][i + d, :])
                        + (bufs[2][i + d, :] + bufs[3][i + d, :]))

        @pl.loop(0, bpw, step=4)
        def _(i):
            for d in range(4):
                acc_v[i + d, :] = jnp.zeros((_LANES,), jnp.float32)

        fire(0, grp_a, sem0)
        fire(4, grp_b, sem1)

        @pl.loop(0, seq, step=8)
        def _(s):
            drain_acc(s, grp_a, sem0)

            @pl.when(s + 8 < seq)
            def _():
                fire(s + 8, grp_a, sem0)

            drain_acc(s + 4, grp_b, sem1)

            @pl.when(s + 12 < seq)
            def _():
                fire(s + 12, grp_b, sem1)

        @pl.loop(0, bpw, step=4)
        def _(i):
            for d in range(4):
                acc_v[i + d, :] = acc_v[i + d, :] + bias_v[:]

        pltpu.sync_copy(acc_v, out_hbm.at[pl.ds(b0, bpw)])

    return pooled


def kernel(text, embedding_table, fc_w, fc_b):
    seq, batch = text.shape
    vocab, dim = embedding_table.shape
    out_dim = fc_w.shape[0]
    pdim = _LANES

    wp = jnp.zeros((dim, pdim), jnp.float32).at[:, :out_dim].set(fc_w.T / seq)
    bias16 = jnp.zeros((pdim,), jnp.float32).at[:out_dim].set(fc_b)

    et = embedding_table.T
    zt = _proj_stage1(vocab, dim, pdim)(wp, et)
    ew128 = _proj_stage2(vocab, pdim)(*([zt] * _PACK))
    ew = ew128.reshape(_PACK * _CHUNK, pdim)
    pooled = _make_pooled(seq, batch, pdim)(text, ew, bias16)
    return pooled[:, :out_dim]


# speedup vs baseline: 7.5733x; 1.3085x over previous
import functools

import jax
import jax.numpy as jnp
from jax import lax
from jax.experimental import pallas as pl
from jax.experimental.pallas import tpu as pltpu
from jax.experimental.pallas import tpu_sc as plsc

_NUM_CORES = 2
_NUM_SUBCORES = 16
_NUM_WORKERS = _NUM_CORES * _NUM_SUBCORES
_LANES = 16
_PACK = 8
_CHUNK_BITS = 17
_CHUNK = 1 << _CHUNK_BITS


def _project(vocab, dim, pdim):
    br = 4096
    nblk = _CHUNK // br
    last_blk = (vocab - 1) // br

    def body(*refs):
        w = refs[_PACK][...]
        o_ref = refs[_PACK + 1]
        zs = [
            jax.lax.dot_general(
                w, refs[j][...], (((0,), (0,)), ((), ())),
                preferred_element_type=jnp.float32)
            for j in range(_PACK)
        ]
        o_ref[...] = jnp.concatenate(zs, axis=0).T

    def make_imap(j):
        def imap(i):
            return (0, jnp.minimum(j * nblk + i, last_blk))
        return imap

    return pl.pallas_call(
        body,
        grid=(nblk,),
        in_specs=[
            pl.BlockSpec((dim, br), make_imap(j)) for j in range(_PACK)
        ] + [pl.BlockSpec((dim, pdim), lambda i: (0, 0))],
        out_specs=pl.BlockSpec((br, _PACK * _LANES), lambda i: (i, 0)),
        out_shape=jax.ShapeDtypeStruct((_CHUNK, _PACK * _LANES), jnp.float32),
    )


def _make_pooled(seq, batch, pdim):
    bpw = batch // _NUM_WORKERS
    mesh = plsc.VectorSubcoreMesh(core_axis_name="c", subcore_axis_name="s")

    @functools.partial(
        pl.kernel,
        mesh=mesh,
        out_type=jax.ShapeDtypeStruct((batch, pdim), jnp.float32),
        compiler_params=pltpu.CompilerParams(use_tc_tiling_on_sc=False),
        scratch_types=[
            pltpu.VMEM((seq, bpw), jnp.int32),
        ] + [pltpu.VMEM((bpw, pdim), jnp.float32) for _ in range(9)] + [
            pltpu.VMEM((_LANES,), jnp.float32),
            pltpu.SemaphoreType.DMA,
            pltpu.SemaphoreType.DMA,
        ],
    )
    def pooled(text_hbm, ew_hbm, bias_hbm, out_hbm, idx_v,
               a0, a1, a2, a3, c0, c1, c2, c3, acc_v, bias_v, sem0, sem1):
        grp_a = (a0, a1, a2, a3)
        grp_b = (c0, c1, c2, c3)
        w = lax.axis_index("s") * _NUM_CORES + lax.axis_index("c")
        b0 = w * bpw

        pltpu.sync_copy(text_hbm.at[:, pl.ds(b0, bpw)], idx_v)
        pltpu.sync_copy(bias_hbm, bias_v)

        @pl.loop(0, seq)
        def _(s):
            for c in range(bpw // _LANES):
                sl = pl.ds(c * _LANES, _LANES)
                i = idx_v[s, sl]
                q = jax.lax.shift_right_logical(i, _CHUNK_BITS)
                r = jax.lax.bitwise_and(i, _CHUNK - 1)
                idx_v[s, sl] = jax.lax.shift_left(r, 3) + q

        def gather(s, buf, sem):
            return pltpu.make_async_copy(ew_hbm.at[idx_v.at[s]], buf, sem)

        def fire(s, bufs, sem):
            for b in range(4):
                gather(s + b, bufs[b], sem).start()

        def drain_acc(s, bufs, sem):
            for b in range(4):
                gather(s + b, bufs[b], sem).wait()

            @pl.loop(0, bpw, step=2)
            def _(i):
                for d in range(2):
                    acc_v[i + d, :] = acc_v[i + d, :] + (
                        (bufs[0][i + d, :] + bufs[1][i + d, :])
                        + (bufs[2][i + d, :] + bufs[3][i + d, :]))

        @pl.loop(0, bpw, step=4)
        def _(i):
            for d in range(4):
                acc_v[i + d, :] = jnp.zeros((_LANES,), jnp.float32)

        fire(0, grp_a, sem0)
        fire(4, grp_b, sem1)

        @pl.loop(0, seq, step=8)
        def _(s):
            drain_acc(s, grp_a, sem0)

            @pl.when(s + 8 < seq)
            def _():
                fire(s + 8, grp_a, sem0)

            drain_acc(s + 4, grp_b, sem1)

            @pl.when(s + 12 < seq)
            def _():
                fire(s + 12, grp_b, sem1)

        @pl.loop(0, bpw, step=4)
        def _(i):
            for d in range(4):
                acc_v[i + d, :] = acc_v[i + d, :] + bias_v[:]

        pltpu.sync_copy(acc_v, out_hbm.at[pl.ds(b0, bpw)])

    return pooled


def kernel(text, embedding_table, fc_w, fc_b):
    seq, batch = text.shape
    vocab, dim = embedding_table.shape
    out_dim = fc_w.shape[0]
    pdim = _LANES

    wp = jnp.zeros((dim, pdim), jnp.float32).at[:, :out_dim].set(fc_w.T / seq)
    bias16 = jnp.zeros((pdim,), jnp.float32).at[:out_dim].set(fc_b)

    et = embedding_table.T
    ew128 = _project(vocab, dim, pdim)(*([et] * _PACK), wp)
    ew = ew128.reshape(_PACK * _CHUNK, pdim)
    pooled = _make_pooled(seq, batch, pdim)(text, ew, bias16)
    return pooled[:, :out_dim]


# speedup vs baseline: 7.8550x; 1.0372x over previous
import functools

import jax
import jax.numpy as jnp
from jax import lax
from jax.experimental import pallas as pl
from jax.experimental.pallas import tpu as pltpu
from jax.experimental.pallas import tpu_sc as plsc

_NUM_CORES = 2
_NUM_SUBCORES = 16
_NUM_WORKERS = _NUM_CORES * _NUM_SUBCORES
_LANES = 16
_PACK = 8
_CHUNK_BITS = 17
_CHUNK = 1 << _CHUNK_BITS


def _project(vocab, dim, pdim):
    br = 8192
    nblk = _CHUNK // br
    last_blk = (vocab - 1) // br

    def body(*refs):
        w = refs[_PACK][...]
        o_ref = refs[_PACK + 1]
        zs = [
            jax.lax.dot_general(
                w, refs[j][...], (((0,), (0,)), ((), ())),
                preferred_element_type=jnp.float32)
            for j in range(_PACK)
        ]
        o_ref[...] = jnp.concatenate(zs, axis=0).T

    def make_imap(j):
        def imap(i):
            return (0, jnp.minimum(j * nblk + i, last_blk))
        return imap

    return pl.pallas_call(
        body,
        grid=(nblk,),
        in_specs=[
            pl.BlockSpec((dim, br), make_imap(j)) for j in range(_PACK)
        ] + [pl.BlockSpec((dim, pdim), lambda i: (0, 0))],
        out_specs=pl.BlockSpec((br, _PACK * _LANES), lambda i: (i, 0)),
        out_shape=jax.ShapeDtypeStruct((_CHUNK, _PACK * _LANES), jnp.float32),
    )


def _make_pooled(seq, batch, pdim):
    bpw = batch // _NUM_WORKERS
    mesh = plsc.VectorSubcoreMesh(core_axis_name="c", subcore_axis_name="s")

    @functools.partial(
        pl.kernel,
        mesh=mesh,
        out_type=jax.ShapeDtypeStruct((batch, pdim), jnp.float32),
        compiler_params=pltpu.CompilerParams(use_tc_tiling_on_sc=False),
        scratch_types=[
            pltpu.VMEM((seq, bpw), jnp.int32),
        ] + [pltpu.VMEM((bpw, pdim), jnp.float32) for _ in range(17)] + [
            pltpu.VMEM((_LANES,), jnp.float32),
            pltpu.SemaphoreType.DMA,
            pltpu.SemaphoreType.DMA,
        ],
    )
    def pooled(text_hbm, ew_hbm, bias_hbm, out_hbm, idx_v,
               a0, a1, a2, a3, a4, a5, a6, a7,
               c0, c1, c2, c3, c4, c5, c6, c7, acc_v, bias_v, sem0, sem1):
        grp_a = (a0, a1, a2, a3, a4, a5, a6, a7)
        grp_b = (c0, c1, c2, c3, c4, c5, c6, c7)
        w = lax.axis_index("s") * _NUM_CORES + lax.axis_index("c")
        b0 = w * bpw

        pltpu.sync_copy(text_hbm.at[:, pl.ds(b0, bpw)], idx_v)
        pltpu.sync_copy(bias_hbm, bias_v)

        @pl.loop(0, seq)
        def _(s):
            for c in range(bpw // _LANES):
                sl = pl.ds(c * _LANES, _LANES)
                i = idx_v[s, sl]
                q = jax.lax.shift_right_logical(i, _CHUNK_BITS)
                r = jax.lax.bitwise_and(i, _CHUNK - 1)
                idx_v[s, sl] = jax.lax.shift_left(r, 3) + q

        def gather(s, buf, sem):
            return pltpu.make_async_copy(ew_hbm.at[idx_v.at[s]], buf, sem)

        def fire(s, bufs, sem):
            for b in range(8):
                gather(s + b, bufs[b], sem).start()

        def drain_acc(s, bufs, sem):
            for b in range(8):
                gather(s + b, bufs[b], sem).wait()

            @pl.loop(0, bpw, step=4)
            def _(i):
                for d in range(4):
                    acc_v[i + d, :] = acc_v[i + d, :] + (
                        ((bufs[0][i + d, :] + bufs[1][i + d, :])
                         + (bufs[2][i + d, :] + bufs[3][i + d, :]))
                        + ((bufs[4][i + d, :] + bufs[5][i + d, :])
                           + (bufs[6][i + d, :] + bufs[7][i + d, :])))

        @pl.loop(0, bpw, step=4)
        def _(i):
            for d in range(4):
                acc_v[i + d, :] = jnp.zeros((_LANES,), jnp.float32)

        fire(0, grp_a, sem0)
        fire(8, grp_b, sem1)

        @pl.loop(0, seq - 8, step=16)
        def _(s):
            drain_acc(s, grp_a, sem0)
            fire(s + 16, grp_a, sem0)
            drain_acc(s + 8, grp_b, sem1)

            @pl.when(s + 24 < seq)
            def _():
                fire(s + 24, grp_b, sem1)

        drain_acc(seq - 8, grp_a, sem0)

        @pl.loop(0, bpw, step=4)
        def _(i):
            for d in range(4):
                acc_v[i + d, :] = acc_v[i + d, :] + bias_v[:]

        pltpu.sync_copy(acc_v, out_hbm.at[pl.ds(b0, bpw)])

    return pooled


def kernel(text, embedding_table, fc_w, fc_b):
    seq, batch = text.shape
    vocab, dim = embedding_table.shape
    out_dim = fc_w.shape[0]
    pdim = _LANES

    wp = jnp.zeros((dim, pdim), jnp.float32).at[:, :out_dim].set(fc_w.T / seq)
    bias16 = jnp.zeros((pdim,), jnp.float32).at[:out_dim].set(fc_b)

    et = embedding_table.T
    ew128 = _project(vocab, dim, pdim)(*([et] * _PACK), wp)
    ew = ew128.reshape(_PACK * _CHUNK, pdim)
    pooled = _make_pooled(seq, batch, pdim)(text, ew, bias16)
    return pooled[:, :out_dim]
